# baseline (speedup 1.0000x reference)
import functools

import jax
import jax.numpy as jnp
from jax import lax
from jax.experimental import pallas as pl
from jax.experimental.pallas import tpu as pltpu
from jax.experimental.pallas import tpu_sc as plsc

NC = 2
NS = 16
L = 16
NW = NC * NS

H = 9
NH = 8
D = 128
RW = H * D

B_SC = 2048
BB_TC = 1024
NB_TC = (16384 - B_SC) // BB_TC


def _sc_slab(x_flat, i_flat, n_rows):
    rows_per_w = n_rows // NW
    ngroups = rows_per_w // L
    gw = L * RW

    mesh = plsc.VectorSubcoreMesh(core_axis_name="c", subcore_axis_name="s")

    @functools.partial(
        pl.kernel,
        out_type=(
            jax.ShapeDtypeStruct((n_rows * NH,), jnp.float32),
            jax.ShapeDtypeStruct((n_rows,), jnp.int32),
            jax.ShapeDtypeStruct((NW * L,), jnp.float32),
            jax.ShapeDtypeStruct((NW * L,), jnp.float32),
        ),
        mesh=mesh,
        scratch_types=[
            pltpu.VMEM((2 * gw,), jnp.float32),
            pltpu.VMEM((rows_per_w * 2,), jnp.int32),
            pltpu.VMEM((rows_per_w * NH,), jnp.float32),
            pltpu.VMEM((rows_per_w,), jnp.int32),
            pltpu.VMEM((L,), jnp.float32),
            pltpu.SemaphoreType.DMA,
        ],
        compiler_params=pltpu.CompilerParams(needs_layout_passes=False),
    )
    def body(x_hbm, i_hbm, diff_hbm, pred_hbm, lpart_hbm, apart_hbm,
             xbuf, ibuf, diffb, predb, stage, sem):
        wid = lax.axis_index("s") * NC + lax.axis_index("c")
        rbase = wid * rows_per_w
        lanes = lax.iota(jnp.int32, L)

        pltpu.sync_copy(i_hbm.at[pl.ds(pl.multiple_of(rbase * 2, 128),
                                       rows_per_w * 2)], ibuf)

        pltpu.async_copy(x_hbm.at[pl.ds(pl.multiple_of(rbase * RW, 2048), gw)],
                         xbuf.at[pl.ds(0, gw)], sem)

        zf = jnp.zeros((L,), jnp.float32)

        def group_body(g, carry):
            loss_vec, acc_vec = carry
            slot = lax.rem(g, 2)
            sbase = slot * gw
            pltpu.make_async_copy(x_hbm.at[pl.ds(0, gw)],
                                  xbuf.at[pl.ds(sbase, gw)], sem).wait()

            @pl.when(g + 1 < ngroups)
            def _():
                off = pl.multiple_of(rbase * RW + (g + 1) * gw, 2048)
                pltpu.async_copy(x_hbm.at[pl.ds(off, gw)],
                                 xbuf.at[pl.ds((1 - slot) * gw, gw)], sem)

            av = lanes * RW + sbase

            def dim_body(d, accs):
                base = av + d
                t = plsc.load_gather(xbuf, [base + NH * D])
                out = []
                for i in range(NH):
                    x = plsc.load_gather(xbuf, [base + i * D])
                    e = x - t
                    out.append(accs[i] + e * e)
                return tuple(out)

            accs = lax.fori_loop(0, D, dim_body, (zf,) * NH, unroll=2)

            yt = plsc.load_gather(ibuf, [g * (2 * L) + lanes * 2]) - 8

            best = accs[0]
            besti = jnp.zeros((L,), jnp.int32)
            rowsum = accs[0]
            sel = jnp.where(yt == 0, accs[0], 0.0)
            for i in range(1, NH):
                a = accs[i]
                lt = a < best
                besti = jnp.where(lt, jnp.int32(i), besti)
                best = jnp.where(lt, a, best)
                rowsum = rowsum + a
                sel = sel + jnp.where(yt == i, a, 0.0)

            loss_vec = loss_vec + (2.0 * sel - rowsum)
            acc_vec = acc_vec + jnp.where(besti == yt, 1.0, 0.0)

            for i in range(NH):
                plsc.store_scatter(diffb, [g * (L * NH) + lanes * NH + i],
                                   accs[i])
            plsc.store_scatter(predb, [g * L + lanes], besti)
            return (loss_vec, acc_vec)

        loss_vec, acc_vec = lax.fori_loop(0, ngroups, group_body, (zf, zf))

        pltpu.sync_copy(diffb, diff_hbm.at[pl.ds(
            pl.multiple_of(rbase * NH, 512), rows_per_w * NH)])
        pltpu.sync_copy(predb, pred_hbm.at[pl.ds(
            pl.multiple_of(rbase, 64), rows_per_w)])
        stage[...] = loss_vec
        pltpu.sync_copy(stage, lpart_hbm.at[pl.ds(
            pl.multiple_of(wid * L, 16), L)])
        stage[...] = acc_vec
        pltpu.sync_copy(stage, apart_hbm.at[pl.ds(
            pl.multiple_of(wid * L, 16), L)])

    return body(x_flat, i_flat)


def _tc_slab(x, y_true):
    bb = BB_TC
    nb = NB_TC
    k0 = B_SC // bb

    def body(x_ref, y_ref, diff_ref, pred_ref, lp_ref, ap_ref):
        t = x_ref[:, NH, :]
        ones = jnp.ones((D, 1), jnp.float32)
        rs = []
        for i in range(NH):
            e = x_ref[:, i, :] - t
            rs.append(jnp.dot(e * e, ones,
                              precision=lax.Precision.HIGHEST)[:, 0])

        yt = y_ref[0, 0, :]
        best = rs[0]
        besti = jnp.zeros((bb,), jnp.int32)
        rowsum = rs[0]
        sel = jnp.where(yt == 0, rs[0], 0.0)
        for i in range(1, NH):
            a = rs[i]
            lt = a < best
            besti = jnp.where(lt, jnp.int32(i), besti)
            best = jnp.where(lt, a, best)
            rowsum = rowsum + a
            sel = sel + jnp.where(yt == i, a, 0.0)

        for i in range(NH):
            diff_ref[i, :] = rs[i]
        pred_ref[0, 0, :] = besti
        lp_ref[...] = jnp.full((1, 1, 128), jnp.sum(2.0 * sel - rowsum),
                               jnp.float32)
        ap_ref[...] = jnp.full((1, 1, 128),
                               jnp.sum(jnp.where(besti == yt, 1.0, 0.0)),
                               jnp.float32)

    return pl.pallas_call(
        body,
        grid=(nb,),
        in_specs=[
            pl.BlockSpec((bb, H, D), lambda k: (k + k0, 0, 0)),
            pl.BlockSpec((1, 1, bb), lambda k: (k + k0, 0, 0)),
        ],
        out_specs=[
            pl.BlockSpec((NH, bb), lambda k: (0, k)),
            pl.BlockSpec((1, 1, bb), lambda k: (k, 0, 0)),
            pl.BlockSpec((1, 1, 128), lambda k: (k, 0, 0)),
            pl.BlockSpec((1, 1, 128), lambda k: (k, 0, 0)),
        ],
        out_shape=(
            jax.ShapeDtypeStruct((NH, nb * bb), jnp.float32),
            jax.ShapeDtypeStruct((nb, 1, bb), jnp.int32),
            jax.ShapeDtypeStruct((nb, 1, 128), jnp.float32),
            jax.ShapeDtypeStruct((nb, 1, 128), jnp.float32),
        ),
    )(x, y_true.reshape(-1, 1, bb))


def _merge(lpart, apart, lp_tc, ap_tc, B):
    mesh = plsc.VectorSubcoreMesh(core_axis_name="c", subcore_axis_name="s")

    @functools.partial(
        pl.kernel,
        out_type=(
            jax.ShapeDtypeStruct((L,), jnp.float32),
            jax.ShapeDtypeStruct((L,), jnp.float32),
        ),
        mesh=mesh,
        scratch_types=[
            pltpu.VMEM((NW * L,), jnp.float32),
            pltpu.VMEM((NW * L,), jnp.float32),
            pltpu.VMEM((2 * L,), jnp.float32),
            pltpu.VMEM((2 * L,), jnp.float32),
            pltpu.VMEM((L,), jnp.float32),
            pltpu.VMEM((L,), jnp.float32),
        ],
        compiler_params=pltpu.CompilerParams(needs_layout_passes=False),
    )
    def body(lpart_hbm, apart_hbm, lptc_hbm, aptc_hbm, loss_out, acc_out,
             lbuf, abuf, ltc, atc, lst, ast):
        wid = lax.axis_index("s") * NC + lax.axis_index("c")

        @pl.when(wid == 0)
        def _():
            pltpu.sync_copy(lpart_hbm, lbuf)
            pltpu.sync_copy(apart_hbm, abuf)
            pltpu.sync_copy(lptc_hbm, ltc)
            pltpu.sync_copy(aptc_hbm, atc)
            ls = ltc[pl.ds(0, L)] + ltc[pl.ds(L, L)]
            ac = atc[pl.ds(0, L)] + atc[pl.ds(L, L)]
            for r in range(NW):
                ls = ls + lbuf[pl.ds(r * L, L)]
                ac = ac + abuf[pl.ds(r * L, L)]
            loss = jnp.sum(ls) * (1.0 / (B * NH))
            acc = jnp.sum(ac) * (1.0 / B)
            lst[...] = jnp.full((L,), loss, jnp.float32)
            ast[...] = jnp.full((L,), acc, jnp.float32)
            pltpu.sync_copy(lst, loss_out)
            pltpu.sync_copy(ast, acc_out)

    return body(lpart, apart, lp_tc, ap_tc)


def kernel(outputs, index):
    B = outputs.shape[0]
    assert outputs.shape[1:] == (H, D)
    assert B_SC % (NW * L) == 0 and B_SC % BB_TC == 0
    assert B == B_SC + NB_TC * BB_TC

    idx32 = index.astype(jnp.int32)
    x_sc = outputs[:B_SC].reshape(-1)
    i_sc = idx32[:B_SC].reshape(-1)
    diff_sc, pred_sc, lpart, apart = _sc_slab(x_sc, i_sc, B_SC)

    y_all = idx32[:, 0] - 8
    diff_t, pred_tc, lp_tc, ap_tc = _tc_slab(outputs, y_all)

    pad = 2 * L - NB_TC
    lp_vec = jnp.pad(lp_tc[:, 0, 0], (0, pad))
    ap_vec = jnp.pad(ap_tc[:, 0, 0], (0, pad))
    loss16, acc16 = _merge(lpart, apart, lp_vec, ap_vec, B)

    diff = jnp.concatenate([diff_sc.reshape(B_SC, NH), diff_t.T], axis=0)
    pred = jnp.concatenate([pred_sc, pred_tc.reshape(-1)], axis=0)
    return diff, pred, loss16[0], acc16[0]


# speedup vs baseline: 1.1712x; 1.1712x over previous
import functools

import jax
import jax.numpy as jnp
from jax import lax
from jax.experimental import pallas as pl
from jax.experimental.pallas import tpu as pltpu
from jax.experimental.pallas import tpu_sc as plsc

NC = 2
NS = 16
L = 16
NW = NC * NS

H = 9
NH = 8
D = 128
RW = H * D

B_SC = 2048
BB_TC = 1024
NB_TC = (16384 - B_SC) // BB_TC


def _sc_slab(x_flat, i_flat, n_rows):
    rows_per_w = n_rows // NW
    ngroups = rows_per_w // L
    gw = L * RW

    mesh = plsc.VectorSubcoreMesh(core_axis_name="c", subcore_axis_name="s")

    @functools.partial(
        pl.kernel,
        out_type=(
            jax.ShapeDtypeStruct((n_rows * NH,), jnp.float32),
            jax.ShapeDtypeStruct((n_rows,), jnp.int32),
            jax.ShapeDtypeStruct((NW * L,), jnp.float32),
            jax.ShapeDtypeStruct((NW * L,), jnp.float32),
        ),
        mesh=mesh,
        scratch_types=[
            pltpu.VMEM((2 * gw,), jnp.float32),
            pltpu.VMEM((rows_per_w * 2,), jnp.int32),
            pltpu.VMEM((rows_per_w * NH,), jnp.float32),
            pltpu.VMEM((rows_per_w,), jnp.int32),
            pltpu.VMEM((L,), jnp.float32),
            pltpu.SemaphoreType.DMA,
        ],
        compiler_params=pltpu.CompilerParams(needs_layout_passes=False),
    )
    def body(x_hbm, i_hbm, diff_hbm, pred_hbm, lpart_hbm, apart_hbm,
             xbuf, ibuf, diffb, predb, stage, sem):
        wid = lax.axis_index("s") * NC + lax.axis_index("c")
        rbase = wid * rows_per_w
        lanes = lax.iota(jnp.int32, L)

        pltpu.sync_copy(i_hbm.at[pl.ds(pl.multiple_of(rbase * 2, 128),
                                       rows_per_w * 2)], ibuf)

        pltpu.async_copy(x_hbm.at[pl.ds(pl.multiple_of(rbase * RW, 2048), gw)],
                         xbuf.at[pl.ds(0, gw)], sem)

        zf = jnp.zeros((L,), jnp.float32)

        def group_body(g, carry):
            loss_vec, acc_vec = carry
            slot = lax.rem(g, 2)
            sbase = slot * gw
            pltpu.make_async_copy(x_hbm.at[pl.ds(0, gw)],
                                  xbuf.at[pl.ds(sbase, gw)], sem).wait()

            @pl.when(g + 1 < ngroups)
            def _():
                off = pl.multiple_of(rbase * RW + (g + 1) * gw, 2048)
                pltpu.async_copy(x_hbm.at[pl.ds(off, gw)],
                                 xbuf.at[pl.ds((1 - slot) * gw, gw)], sem)

            av = lanes * RW + sbase

            def dim_body(d, accs):
                base = av + d
                t = plsc.load_gather(xbuf, [base + NH * D])
                out = []
                for i in range(NH):
                    x = plsc.load_gather(xbuf, [base + i * D])
                    e = x - t
                    out.append(accs[i] + e * e)
                return tuple(out)

            accs = lax.fori_loop(0, D, dim_body, (zf,) * NH, unroll=2)

            yt = plsc.load_gather(ibuf, [g * (2 * L) + lanes * 2]) - 8

            best = accs[0]
            besti = jnp.zeros((L,), jnp.int32)
            rowsum = accs[0]
            sel = jnp.where(yt == 0, accs[0], 0.0)
            for i in range(1, NH):
                a = accs[i]
                lt = a < best
                besti = jnp.where(lt, jnp.int32(i), besti)
                best = jnp.where(lt, a, best)
                rowsum = rowsum + a
                sel = sel + jnp.where(yt == i, a, 0.0)

            loss_vec = loss_vec + (2.0 * sel - rowsum)
            acc_vec = acc_vec + jnp.where(besti == yt, 1.0, 0.0)

            for i in range(NH):
                plsc.store_scatter(diffb, [g * (L * NH) + lanes * NH + i],
                                   accs[i])
            plsc.store_scatter(predb, [g * L + lanes], besti)
            return (loss_vec, acc_vec)

        loss_vec, acc_vec = lax.fori_loop(0, ngroups, group_body, (zf, zf))

        pltpu.sync_copy(diffb, diff_hbm.at[pl.ds(
            pl.multiple_of(rbase * NH, 512), rows_per_w * NH)])
        pltpu.sync_copy(predb, pred_hbm.at[pl.ds(
            pl.multiple_of(rbase, 64), rows_per_w)])
        stage[...] = loss_vec
        pltpu.sync_copy(stage, lpart_hbm.at[pl.ds(
            pl.multiple_of(wid * L, 16), L)])
        stage[...] = acc_vec
        pltpu.sync_copy(stage, apart_hbm.at[pl.ds(
            pl.multiple_of(wid * L, 16), L)])

    return body(x_flat, i_flat)


def _tc_slab(x, y_true):
    bb = BB_TC
    nb = NB_TC
    k0 = B_SC // bb

    def body(x_ref, y_ref, diff_ref, pred_ref, lp_ref, ap_ref):
        t = x_ref[:, NH, :]
        rs = []
        for i in range(NH):
            e = x_ref[:, i, :] - t
            rs.append(jnp.sum(e * e, axis=-1))

        yt = y_ref[0, 0, :]
        best = rs[0]
        besti = jnp.zeros((bb,), jnp.int32)
        rowsum = rs[0]
        sel = jnp.where(yt == 0, rs[0], 0.0)
        for i in range(1, NH):
            a = rs[i]
            lt = a < best
            besti = jnp.where(lt, jnp.int32(i), besti)
            best = jnp.where(lt, a, best)
            rowsum = rowsum + a
            sel = sel + jnp.where(yt == i, a, 0.0)

        for i in range(NH):
            diff_ref[i, :] = rs[i]
        pred_ref[0, 0, :] = besti
        lp_ref[...] = jnp.full((1, 1, 128), jnp.sum(2.0 * sel - rowsum),
                               jnp.float32)
        ap_ref[...] = jnp.full((1, 1, 128),
                               jnp.sum(jnp.where(besti == yt, 1.0, 0.0)),
                               jnp.float32)

    return pl.pallas_call(
        body,
        grid=(nb,),
        in_specs=[
            pl.BlockSpec((bb, H, D), lambda k: (k + k0, 0, 0)),
            pl.BlockSpec((1, 1, bb), lambda k: (k + k0, 0, 0)),
        ],
        out_specs=[
            pl.BlockSpec((NH, bb), lambda k: (0, k)),
            pl.BlockSpec((1, 1, bb), lambda k: (k, 0, 0)),
            pl.BlockSpec((1, 1, 128), lambda k: (k, 0, 0)),
            pl.BlockSpec((1, 1, 128), lambda k: (k, 0, 0)),
        ],
        out_shape=(
            jax.ShapeDtypeStruct((NH, nb * bb), jnp.float32),
            jax.ShapeDtypeStruct((nb, 1, bb), jnp.int32),
            jax.ShapeDtypeStruct((nb, 1, 128), jnp.float32),
            jax.ShapeDtypeStruct((nb, 1, 128), jnp.float32),
        ),
    )(x, y_true.reshape(-1, 1, bb))


def _merge(lpart, apart, lp_tc, ap_tc, B):
    mesh = plsc.VectorSubcoreMesh(core_axis_name="c", subcore_axis_name="s")

    @functools.partial(
        pl.kernel,
        out_type=(
            jax.ShapeDtypeStruct((L,), jnp.float32),
            jax.ShapeDtypeStruct((L,), jnp.float32),
        ),
        mesh=mesh,
        scratch_types=[
            pltpu.VMEM((NW * L,), jnp.float32),
            pltpu.VMEM((NW * L,), jnp.float32),
            pltpu.VMEM((2 * L,), jnp.float32),
            pltpu.VMEM((2 * L,), jnp.float32),
            pltpu.VMEM((L,), jnp.float32),
            pltpu.VMEM((L,), jnp.float32),
        ],
        compiler_params=pltpu.CompilerParams(needs_layout_passes=False),
    )
    def body(lpart_hbm, apart_hbm, lptc_hbm, aptc_hbm, loss_out, acc_out,
             lbuf, abuf, ltc, atc, lst, ast):
        wid = lax.axis_index("s") * NC + lax.axis_index("c")

        @pl.when(wid == 0)
        def _():
            pltpu.sync_copy(lpart_hbm, lbuf)
            pltpu.sync_copy(apart_hbm, abuf)
            pltpu.sync_copy(lptc_hbm, ltc)
            pltpu.sync_copy(aptc_hbm, atc)
            ls = ltc[pl.ds(0, L)] + ltc[pl.ds(L, L)]
            ac = atc[pl.ds(0, L)] + atc[pl.ds(L, L)]
            for r in range(NW):
                ls = ls + lbuf[pl.ds(r * L, L)]
                ac = ac + abuf[pl.ds(r * L, L)]
            loss = jnp.sum(ls) * (1.0 / (B * NH))
            acc = jnp.sum(ac) * (1.0 / B)
            lst[...] = jnp.full((L,), loss, jnp.float32)
            ast[...] = jnp.full((L,), acc, jnp.float32)
            pltpu.sync_copy(lst, loss_out)
            pltpu.sync_copy(ast, acc_out)

    return body(lpart, apart, lp_tc, ap_tc)


def kernel(outputs, index):
    B = outputs.shape[0]
    assert outputs.shape[1:] == (H, D)
    assert B_SC % (NW * L) == 0 and B_SC % BB_TC == 0
    assert B == B_SC + NB_TC * BB_TC

    idx32 = index.astype(jnp.int32)
    x_sc = outputs[:B_SC].reshape(-1)
    i_sc = idx32[:B_SC].reshape(-1)
    diff_sc, pred_sc, lpart, apart = _sc_slab(x_sc, i_sc, B_SC)

    y_all = idx32[:, 0] - 8
    diff_t, pred_tc, lp_tc, ap_tc = _tc_slab(outputs, y_all)

    pad = 2 * L - NB_TC
    lp_vec = jnp.pad(lp_tc[:, 0, 0], (0, pad))
    ap_vec = jnp.pad(ap_tc[:, 0, 0], (0, pad))
    loss16, acc16 = _merge(lpart, apart, lp_vec, ap_vec, B)

    diff = jnp.concatenate([diff_sc.reshape(B_SC, NH), diff_t.T], axis=0)
    pred = jnp.concatenate([pred_sc, pred_tc.reshape(-1)], axis=0)
    return diff, pred, loss16[0], acc16[0]
